# baseline (speedup 1.0000x reference)
import functools

import jax
import jax.numpy as jnp
from jax import lax
from jax.experimental import pallas as pl
from jax.experimental.pallas import tpu as pltpu
from jax.experimental.pallas import tpu_sc as plsc

_L = 16
_TPAD = 129


def _make_emb_kernel(NW, NC, n_groups_per_w, D, NB):
    mesh = plsc.VectorSubcoreMesh(core_axis_name="c", subcore_axis_name="s")
    CH = 128
    DB = D // 8

    @functools.partial(
        pl.kernel,
        mesh=mesh,
        out_type=jax.ShapeDtypeStruct(
            (NW * n_groups_per_w // NB, DB, NB, 8, CH), jnp.float32
        ),
        scratch_types=[
            pltpu.VMEM((n_groups_per_w, CH), jnp.int32),
            pltpu.VMEM((CH, D), jnp.float32),
            pltpu.VMEM((CH, D), jnp.float32),
            pltpu.VMEM((D, _TPAD), jnp.float32),
            pltpu.VMEM((D, _TPAD), jnp.float32),
            pltpu.SemaphoreType.DMA,
            pltpu.SemaphoreType.DMA,
            pltpu.SemaphoreType.DMA,
            pltpu.SemaphoreType.DMA,
        ],
        compiler_params=pltpu.CompilerParams(
            use_tc_tiling_on_sc=False, needs_layout_passes=False
        ),
    )
    def emb(table_hbm, idx_hbm, out_hbm, idx_v, rowA, rowB, tilA, tilB,
            gsA, gsB, osA, osB):
        wid = lax.axis_index("s") * NC + lax.axis_index("c")
        gbase = wid * n_groups_per_w
        pltpu.sync_copy(idx_hbm.at[wid], idx_v)

        lane = lax.broadcasted_iota(jnp.int32, (_L,), 0)
        zeros = lane - lane
        d_idx = [lane + j * _L for j in range(D // _L)]

        def transpose(row, til):
            @plsc.parallel_loop(0, CH, step=1, unroll=16)
            def _(t):
                t_idx = zeros + t
                for j in range(D // _L):
                    v = row[t, pl.ds(j * _L, _L)]
                    plsc.store_scatter(til, [d_idx[j], t_idx], v)

        def fire_scatters(gl, til, sem):
            g = gbase + gl
            s = g // NB
            bblk = g - s * NB
            for db in range(DB):
                pltpu.make_async_copy(
                    til.at[pl.ds(db * 8, 8), pl.ds(0, CH)],
                    out_hbm.at[s, db, bblk],
                    sem,
                ).start()

        def drain_scatters(til, sem):
            for db in range(DB):
                pltpu.make_async_copy(
                    til.at[pl.ds(db * 8, 8), pl.ds(0, CH)],
                    out_hbm.at[0, 0, 0],
                    sem,
                ).wait()

        n_pairs = n_groups_per_w // 2

        def pair(p, carry):
            gA = 2 * p

            @pl.when(p > 0)
            def _():
                drain_scatters(tilA, osA)

            hA = pltpu.make_async_copy(
                table_hbm.at[idx_v.at[gA]], rowA, gsA
            )
            hA.start()

            @pl.when(p > 0)
            def _():
                drain_scatters(tilB, osB)

            hB = pltpu.make_async_copy(
                table_hbm.at[idx_v.at[gA + 1]], rowB, gsB
            )
            hB.start()
            hA.wait()
            transpose(rowA, tilA)
            fire_scatters(gA, tilA, osA)
            hB.wait()
            transpose(rowB, tilB)
            fire_scatters(gA + 1, tilB, osB)
            return carry

        lax.fori_loop(0, n_pairs, pair, 0)
        drain_scatters(tilA, osA)
        drain_scatters(tilB, osB)

    return emb


def kernel(x, table):
    B, S = x.shape
    V, D = table.shape
    info = plsc.get_sparse_core_info()
    NC, NS = info.num_cores, info.num_subcores
    NW = NC * NS
    CH = 128
    NB = B // CH
    n_groups = S * NB
    n_groups_per_w = n_groups // NW
    assert n_groups_per_w * NW == n_groups
    assert n_groups_per_w % 2 == 0

    idx = jnp.transpose(x).reshape(NW, n_groups_per_w, CH).astype(jnp.int32)
    emb = _make_emb_kernel(NW, NC, n_groups_per_w, D, NB)
    out5 = emb(table, idx)
    return jnp.transpose(out5, (2, 4, 0, 1, 3)).reshape(B, S, D)


# speedup vs baseline: 1.0101x; 1.0101x over previous
import functools

import jax
import jax.numpy as jnp
from jax import lax
from jax.experimental import pallas as pl
from jax.experimental.pallas import tpu as pltpu
from jax.experimental.pallas import tpu_sc as plsc

_L = 16
_TPAD = 129


def _make_emb_kernel(NW, NC, n_groups_per_w, D, NB):
    mesh = plsc.VectorSubcoreMesh(core_axis_name="c", subcore_axis_name="s")
    CH = 128
    DB = D // 8

    @functools.partial(
        pl.kernel,
        mesh=mesh,
        out_type=jax.ShapeDtypeStruct(
            (NW * n_groups_per_w // NB, DB, NB, 8, CH), jnp.float32
        ),
        scratch_types=[
            pltpu.VMEM((n_groups_per_w, CH), jnp.int32),
            pltpu.VMEM((CH, D), jnp.float32),
            pltpu.VMEM((CH, D), jnp.float32),
            pltpu.VMEM((D, _TPAD), jnp.float32),
            pltpu.VMEM((D, _TPAD), jnp.float32),
            pltpu.SemaphoreType.DMA,
            pltpu.SemaphoreType.DMA,
            pltpu.SemaphoreType.DMA,
            pltpu.SemaphoreType.DMA,
        ],
        compiler_params=pltpu.CompilerParams(
            use_tc_tiling_on_sc=False, needs_layout_passes=False
        ),
    )
    def emb(table_hbm, idx_hbm, out_hbm, idx_v, rowA, rowB, tilA, tilB,
            gsA, gsB, osA, osB):
        wid = lax.axis_index("s") * NC + lax.axis_index("c")
        gbase = wid * n_groups_per_w
        pltpu.sync_copy(idx_hbm.at[wid], idx_v)

        lane = lax.broadcasted_iota(jnp.int32, (_L,), 0)
        zeros = lane - lane
        d_idx = [lane + j * _L for j in range(D // _L)]

        def transpose(row, til):
            @plsc.parallel_loop(0, CH, step=1, unroll=8)
            def _(t):
                t_idx = zeros + t
                for j in range(D // _L):
                    v = row[t, pl.ds(j * _L, _L)]
                    plsc.store_scatter(til, [d_idx[j], t_idx], v)

        def fire_scatters(gl, til, sem):
            g = gbase + gl
            s = g // NB
            bblk = g - s * NB
            for db in range(DB):
                pltpu.make_async_copy(
                    til.at[pl.ds(db * 8, 8), pl.ds(0, CH)],
                    out_hbm.at[s, db, bblk],
                    sem,
                ).start()

        def drain_scatters(til, sem):
            for db in range(DB):
                pltpu.make_async_copy(
                    til.at[pl.ds(db * 8, 8), pl.ds(0, CH)],
                    out_hbm.at[0, 0, 0],
                    sem,
                ).wait()

        n_pairs = n_groups_per_w // 2

        def pair(p, carry):
            gA = 2 * p

            @pl.when(p > 0)
            def _():
                drain_scatters(tilA, osA)

            hA = pltpu.make_async_copy(
                table_hbm.at[idx_v.at[gA]], rowA, gsA
            )
            hA.start()

            @pl.when(p > 0)
            def _():
                drain_scatters(tilB, osB)

            hB = pltpu.make_async_copy(
                table_hbm.at[idx_v.at[gA + 1]], rowB, gsB
            )
            hB.start()
            hA.wait()
            transpose(rowA, tilA)
            fire_scatters(gA, tilA, osA)
            hB.wait()
            transpose(rowB, tilB)
            fire_scatters(gA + 1, tilB, osB)
            return carry

        lax.fori_loop(0, n_pairs, pair, 0)
        drain_scatters(tilA, osA)
        drain_scatters(tilB, osB)

    return emb


def kernel(x, table):
    B, S = x.shape
    V, D = table.shape
    info = plsc.get_sparse_core_info()
    NC, NS = info.num_cores, info.num_subcores
    NW = NC * NS
    CH = 128
    NB = B // CH
    n_groups = S * NB
    n_groups_per_w = n_groups // NW
    assert n_groups_per_w * NW == n_groups
    assert n_groups_per_w % 2 == 0

    idx = jnp.transpose(x).reshape(NW, n_groups_per_w, CH).astype(jnp.int32)
    emb = _make_emb_kernel(NW, NC, n_groups_per_w, D, NB)
    out5 = emb(table, idx)
    return jnp.transpose(out5, (2, 4, 0, 1, 3)).reshape(B, S, D)


# speedup vs baseline: 1.0846x; 1.0737x over previous
import functools

import jax
import jax.numpy as jnp
from jax import lax
from jax.experimental import pallas as pl
from jax.experimental.pallas import tpu as pltpu
from jax.experimental.pallas import tpu_sc as plsc

_L = 16
_TPAD = 129


def _make_emb_kernel(NW, NC, n_groups_per_w, D, NB):
    mesh = plsc.VectorSubcoreMesh(core_axis_name="c", subcore_axis_name="s")
    CH = 128
    DB = D // 8

    @functools.partial(
        pl.kernel,
        mesh=mesh,
        out_type=jax.ShapeDtypeStruct(
            (NW * n_groups_per_w // NB, DB, NB, 8, CH), jnp.float32
        ),
        scratch_types=[
            pltpu.VMEM((n_groups_per_w, CH), jnp.int32),
            pltpu.VMEM((CH, D), jnp.float32),
            pltpu.VMEM((CH, D), jnp.float32),
            pltpu.VMEM((D, _TPAD), jnp.float32),
            pltpu.VMEM((D, _TPAD), jnp.float32),
            pltpu.SemaphoreType.DMA,
            pltpu.SemaphoreType.DMA,
            pltpu.SemaphoreType.DMA,
            pltpu.SemaphoreType.DMA,
        ],
        compiler_params=pltpu.CompilerParams(
            use_tc_tiling_on_sc=False, needs_layout_passes=False
        ),
    )
    def emb(table_hbm, idx_hbm, out_hbm, idx_v, rowA, rowB, tilA, tilB,
            gsA, gsB, osA, osB):
        wid = lax.axis_index("s") * NC + lax.axis_index("c")
        gbase = wid * n_groups_per_w
        pltpu.sync_copy(idx_hbm.at[wid], idx_v)

        lane = lax.broadcasted_iota(jnp.int32, (_L,), 0)
        zeros = lane - lane
        d_idx = [lane + j * _L for j in range(D // _L)]

        def transpose(row, til):
            @plsc.parallel_loop(0, CH, step=1, unroll=8)
            def _(t):
                t_idx = zeros + t
                for j in range(D // _L):
                    v = row[t, pl.ds(j * _L, _L)]
                    plsc.store_scatter(til, [d_idx[j], t_idx], v)

        def fire_scatters(gl, til, sem):
            g = gbase + gl
            s = g // NB
            bblk = g - s * NB
            for db in range(DB):
                pltpu.make_async_copy(
                    til.at[pl.ds(db * 8, 8), pl.ds(0, CH)],
                    out_hbm.at[s, db, bblk],
                    sem,
                ).start()

        def fire_gather(gl, row, sem):
            pltpu.make_async_copy(
                table_hbm.at[idx_v.at[gl]], row, sem
            ).start()

        def drain_gather(row, sem):
            pltpu.make_async_copy(
                table_hbm.at[idx_v.at[0]], row, sem
            ).wait()

        def drain_scatters(til, sem):
            for db in range(DB):
                pltpu.make_async_copy(
                    til.at[pl.ds(db * 8, 8), pl.ds(0, CH)],
                    out_hbm.at[0, 0, 0],
                    sem,
                ).wait()

        n_pairs = n_groups_per_w // 2

        fire_gather(0, rowA, gsA)
        fire_gather(1, rowB, gsB)

        def pair(p, carry):
            gA = 2 * p

            @pl.when(p > 0)
            def _():
                drain_scatters(tilA, osA)

            drain_gather(rowA, gsA)
            transpose(rowA, tilA)

            @pl.when(p < n_pairs - 1)
            def _():
                fire_gather(gA + 2, rowA, gsA)

            fire_scatters(gA, tilA, osA)

            @pl.when(p > 0)
            def _():
                drain_scatters(tilB, osB)

            drain_gather(rowB, gsB)
            transpose(rowB, tilB)

            @pl.when(p < n_pairs - 1)
            def _():
                fire_gather(gA + 3, rowB, gsB)

            fire_scatters(gA + 1, tilB, osB)
            return carry

        lax.fori_loop(0, n_pairs, pair, 0)
        drain_scatters(tilA, osA)
        drain_scatters(tilB, osB)

    return emb


def kernel(x, table):
    B, S = x.shape
    V, D = table.shape
    info = plsc.get_sparse_core_info()
    NC, NS = info.num_cores, info.num_subcores
    NW = NC * NS
    CH = 128
    NB = B // CH
    n_groups = S * NB
    n_groups_per_w = n_groups // NW
    assert n_groups_per_w * NW == n_groups
    assert n_groups_per_w % 2 == 0

    idx = jnp.transpose(x).reshape(NW, n_groups_per_w, CH).astype(jnp.int32)
    emb = _make_emb_kernel(NW, NC, n_groups_per_w, D, NB)
    out5 = emb(table, idx)
    return jnp.transpose(out5, (2, 4, 0, 1, 3)).reshape(B, S, D)
